# Initial kernel scaffold; baseline (speedup 1.0000x reference)
import jax
import jax.numpy as jnp
from jax.experimental import pallas as pl


def kernel(user_id, item_id, user_bias_score, item_bias_score):
    raise NotImplementedError("write your pallas kernel here")


# speedup vs baseline: 1.3880x; 1.3880x over previous
import functools

import jax
import jax.numpy as jnp
from jax import lax
from jax.experimental import pallas as pl
from jax.experimental.pallas import tpu as pltpu
from jax.experimental.pallas import tpu_sc as plsc

_GLOBAL_AVG = 3.5
_BATCH = 16384


@jax.jit
def kernel(user_id, item_id, user_bias_score, item_bias_score):
    info = plsc.get_sparse_core_info()
    nc, ns, lanes = info.num_cores, info.num_subcores, info.num_lanes
    nw = nc * ns
    b_per_w = _BATCH // nw

    mesh = plsc.VectorSubcoreMesh(core_axis_name="c", subcore_axis_name="s")

    @functools.partial(
        pl.kernel,
        out_type=jax.ShapeDtypeStruct((_BATCH,), jnp.float32),
        mesh=mesh,
        scratch_types=[
            pltpu.VMEM((b_per_w,), jnp.int32),
            pltpu.VMEM((b_per_w,), jnp.int32),
            pltpu.VMEM((b_per_w,), jnp.float32),
            pltpu.VMEM((b_per_w,), jnp.float32),
            pltpu.SemaphoreType.DMA,
            pltpu.SemaphoreType.DMA,
        ],
    )
    def run(uid_hbm, iid_hbm, utab_hbm, itab_hbm, out_hbm,
            uidx_v, iidx_v, uval_v, ival_v, sem_u, sem_i):
        wid = lax.axis_index("s") * nc + lax.axis_index("c")
        base = wid * b_per_w

        cp_u = pltpu.async_copy(uid_hbm.at[pl.ds(base, b_per_w)], uidx_v, sem_u)
        cp_i = pltpu.async_copy(iid_hbm.at[pl.ds(base, b_per_w)], iidx_v, sem_i)
        cp_u.wait()
        g_u = pltpu.async_copy(utab_hbm.at[uidx_v], uval_v, sem_u)
        cp_i.wait()
        g_i = pltpu.async_copy(itab_hbm.at[iidx_v], ival_v, sem_i)
        g_u.wait()
        g_i.wait()

        @pl.loop(0, b_per_w, step=lanes)
        def _(j):
            s = pl.ds(j, lanes)
            uval_v[s] = uval_v[s] + ival_v[s] + _GLOBAL_AVG

        pltpu.sync_copy(uval_v, out_hbm.at[pl.ds(base, b_per_w)])

    return run(user_id, item_id, user_bias_score, item_bias_score)
